# baseline (speedup 1.0000x reference)
import functools
import jax
import jax.numpy as jnp
import numpy as np
from jax import lax
from jax.experimental import pallas as pl
from jax.experimental.pallas import tpu as pltpu
from jax.experimental.pallas import tpu_sc as plsc

_IDX = np.array(
    [3, 17, 29, 42, 56, 61, 73, 88, 91, 104, 111, 123, 130, 142, 150,
     158, 163, 171, 180, 187, 195, 7, 12, 25, 33, 47, 52, 66, 79, 83,
     96, 101, 115, 127, 135, 146, 153, 167, 174, 182, 190, 199, 5, 19,
     38, 59, 70, 99, 119, 139],
    dtype=np.int32,
)

_B, _R, _F = 16384, 200, 64
_K = _IDX.shape[0]
_NW = 32
_ROWS_PER_W = _B * _K // _NW
_CHUNK = 1024
_NCHUNK = _ROWS_PER_W // _CHUNK

_FLAT_IDX = (np.arange(_B, dtype=np.int32)[:, None] * _R
             + _IDX[None, :]).reshape(-1)


def kernel(x):
    xf = x.reshape(_B * _R, _F)
    idx = jnp.asarray(_FLAT_IDX)
    mesh = plsc.VectorSubcoreMesh(core_axis_name="c", subcore_axis_name="s")

    @functools.partial(
        pl.kernel,
        mesh=mesh,
        out_type=jax.ShapeDtypeStruct((_B * _K, _F), jnp.float32),
        scratch_types=[
            pltpu.VMEM((_CHUNK,), jnp.int32),
            pltpu.VMEM((_CHUNK, _F), jnp.float32),
            pltpu.SemaphoreType.DMA,
        ],
        compiler_params=pltpu.CompilerParams(use_tc_tiling_on_sc=False),
    )
    def sc_gather(x_hbm, idx_hbm, out_hbm, idx_v, rows_v, sem):
        wid = lax.axis_index("s") * 2 + lax.axis_index("c")
        base_w = wid * _ROWS_PER_W

        def chunk_body(i, carry):
            base = base_w + i * _CHUNK
            pltpu.sync_copy(idx_hbm.at[pl.ds(base, _CHUNK)], idx_v)
            pltpu.async_copy(x_hbm.at[idx_v], rows_v, sem).wait()
            pltpu.sync_copy(rows_v, out_hbm.at[pl.ds(base, _CHUNK)])
            return carry

        lax.fori_loop(0, _NCHUNK, chunk_body, 0)

    out = sc_gather(xf, idx)
    return out.reshape(_B, _K, _F)


# speedup vs baseline: 19.1840x; 19.1840x over previous
import jax
import jax.numpy as jnp
import numpy as np
from jax.experimental import pallas as pl
from jax.experimental.pallas import tpu as pltpu

_IDX = np.array(
    [3, 17, 29, 42, 56, 61, 73, 88, 91, 104, 111, 123, 130, 142, 150,
     158, 163, 171, 180, 187, 195, 7, 12, 25, 33, 47, 52, 66, 79, 83,
     96, 101, 115, 127, 135, 146, 153, 167, 174, 182, 190, 199, 5, 19,
     38, 59, 70, 99, 119, 139],
    dtype=np.int32,
)
_K = _IDX.shape[0]


def _copy_body(idx_ref, x_ref, o_ref):
    o_ref[...] = x_ref[...]


def kernel(x):
    B, R, F = x.shape
    xt = jnp.transpose(x, (1, 2, 0))
    idx = jnp.asarray(_IDX)

    out_t = pl.pallas_call(
        _copy_body,
        grid_spec=pltpu.PrefetchScalarGridSpec(
            num_scalar_prefetch=1,
            grid=(_K,),
            in_specs=[
                pl.BlockSpec((1, F, B), lambda j, idx_ref: (idx_ref[j], 0, 0)),
            ],
            out_specs=pl.BlockSpec((1, F, B), lambda j, idx_ref: (j, 0, 0)),
        ),
        out_shape=jax.ShapeDtypeStruct((_K, F, B), x.dtype),
    )(idx, xt)
    return out_t.transpose(2, 0, 1)
